# baseline (speedup 1.0000x reference)
import functools

import jax
import jax.numpy as jnp
from jax import lax
from jax.experimental import pallas as pl
from jax.experimental.pallas import tpu as pltpu, tpu_sc as plsc

FIELDS = 26
VOCAB = 100000
EMB = 16
B = 4096
EPS = 1e-5

NC = 2
NS = 16
NW = NC * NS
BPW = B // NW
NG = BPW // 16
PAIRS_W = BPW * FIELDS

def _make_sc_kernel(interpret=False):
    mesh = plsc.VectorSubcoreMesh(core_axis_name="c", subcore_axis_name="s")
    return functools.partial(
        pl.kernel,
        mesh=mesh,
        compiler_params=pltpu.CompilerParams(use_tc_tiling_on_sc=False),
        out_type=jax.ShapeDtypeStruct((B,), jnp.float32),
        interpret=interpret,
        scratch_types=[
            pltpu.VMEM((PAIRS_W // 128, 128), jnp.int32),
            pltpu.VMEM((PAIRS_W, EMB), jnp.float32),
            pltpu.VMEM((PAIRS_W // 128, 128), jnp.int32),
            pltpu.VMEM((PAIRS_W,), jnp.float32),
            pltpu.VMEM((PAIRS_W,), jnp.float32),
            pltpu.VMEM((FIELDS * EMB,), jnp.float32),
            pltpu.VMEM((16,), jnp.float32),
            pltpu.VMEM((BPW,), jnp.float32),
            pltpu.SemaphoreType.DMA,
            pltpu.SemaphoreType.DMA,
        ],
    )(_deepfm_body)


def _deepfm_body(fm2_hbm, fm1_hbm, idx2_hbm, idxT_hbm, xvT_hbm, w_hbm, c_hbm,
               out_hbm, idx2_v, rows_v, idxT_v, fm1g_v, xvT_v, w_v, c_v,
               out_v, sem_a, sem_b):
    wid = lax.axis_index("s") * NC + lax.axis_index("c")
    pair0 = wid * PAIRS_W
    nchunks = PAIRS_W // 128

    pltpu.sync_copy(idx2_hbm.at[pl.ds(wid * nchunks, nchunks)], idx2_v)
    pltpu.sync_copy(idxT_hbm.at[pl.ds(wid * nchunks, nchunks)], idxT_v)
    g2s = [
        pltpu.async_copy(fm2_hbm.at[idx2_v.at[j]],
                         rows_v.at[pl.ds(j * 128, 128)], sem_a)
        for j in range(nchunks)
    ]
    g1s = [
        pltpu.async_copy(fm1_hbm.at[idxT_v.at[j]],
                         fm1g_v.at[pl.ds(j * 128, 128)], sem_b)
        for j in range(nchunks)
    ]
    pltpu.sync_copy(xvT_hbm.at[pl.ds(pair0, PAIRS_W)], xvT_v)
    pltpu.sync_copy(w_hbm, w_v)
    pltpu.sync_copy(c_hbm, c_v)
    for h in g2s:
        h.wait()
    for h in g1s:
        h.wait()

    lanes = lax.iota(jnp.int32, 16)
    cvec = c_v[...]

    def _perm(vec, idx):
        return jax.lax.gather(
            vec, idx[:, None],
            jax.lax.GatherDimensionNumbers(
                offset_dims=(), collapsed_slice_dims=(0,),
                start_index_map=(0,)),
            (1,), mode=jax.lax.GatherScatterMode.PROMISE_IN_BOUNDS)

    def _lane_bcast(vec, i):
        return _perm(vec, jnp.full((16,), i, jnp.int32))

    bfly = [lanes ^ sh for sh in (8, 4, 2, 1)]

    def _lane_sum_all(v):
        for idx in bfly:
            v = v + _perm(v, idx)
        return v

    for g in range(NG):
        first = cvec
        for f in range(FIELDS):
            t0 = (g * FIELDS + f) * 16
            first = first + fm1g_v[pl.ds(t0, 16)] * xvT_v[pl.ds(t0, 16)]

        def bbody(bl, acc, g=g):
            kbase = (g * 16 + bl) * FIELDS
            S = jnp.zeros((16,), jnp.float32)
            Q = jnp.zeros((16,), jnp.float32)
            L = jnp.zeros((16,), jnp.float32)
            for f in range(FIELDS):
                row = rows_v[kbase + f, :]
                xvrow = xvT_v[pl.ds((g * FIELDS + f) * 16, 16)]
                xv = _lane_bcast(xvrow, bl)
                xe = xv * row
                S = S + xe
                Q = Q + xe * xe
                L = L + xe * w_v[pl.ds(f * EMB, 16)]
            val = _lane_sum_all(0.5 * (S * S - Q) + L)
            return jnp.where(lanes == bl, val, acc)

        res = lax.fori_loop(0, 16, bbody, jnp.zeros((16,), jnp.float32))
        out_v[pl.ds(g * 16, 16)] = res + first

    pltpu.sync_copy(out_v, out_hbm.at[pl.ds(wid * BPW, BPW)])


_sc_kernel_cache = None


def _get_sc_kernel():
    global _sc_kernel_cache
    if _sc_kernel_cache is None:
        _sc_kernel_cache = _make_sc_kernel()
    return _sc_kernel_cache


def kernel(Xi, Xv, fm1, fm2, W1, b1, g1, be1, W2, b2, g2, be2, bias):
    s = 1.0 / jnp.sqrt(1.0 + EPS)
    u = (s * g2) @ W2
    a1 = s * g1 * u
    w_eff = W1.T @ a1
    const = b1 @ a1 + be1 @ u + b2 @ (s * g2) + be2.sum() + bias[0]
    c_arr = jnp.broadcast_to(const, (16,)).astype(jnp.float32)

    idx = Xi[:, :, 0].astype(jnp.int32)
    flat = idx + (jnp.arange(FIELDS, dtype=jnp.int32) * VOCAB)[None, :]
    idx2 = flat.reshape(B * FIELDS // 128, 128)
    idxT = flat.reshape(B // 16, 16, FIELDS).transpose(0, 2, 1) \
               .reshape(B * FIELDS // 128, 128)
    xvT = Xv.astype(jnp.float32).reshape(B // 16, 16, FIELDS) \
            .transpose(0, 2, 1).reshape(-1)

    fm2_flat = fm2.reshape(FIELDS * VOCAB, EMB)
    fm1_flat = fm1.reshape(FIELDS * VOCAB)

    return _get_sc_kernel()(fm2_flat, fm1_flat, idx2, idxT, xvT,
                            w_eff.astype(jnp.float32), c_arr)
